# baseline (device time: 357472 ns/iter reference)
import jax
import jax.numpy as jnp
from jax import lax
from jax.experimental import pallas as pl
from jax.experimental.pallas import tpu as pltpu

N_DEV = 4
SQ = 1024
SKV = 1024
HQ_LOCAL = 8
DH = 128
D_MODEL = 1024
BLK = 64
SCALE = 0.08838834764831843
NEG = -1e9


def kernel(x, Wq, K_ext, V_ext, Wo):
    def body(x_ref, wq_ref, k_ref, v_ref, wo_ref, out_ref,
             kv_buf, comm,
             kv_send_sems, kv_recv_sems, kv_local_sems,
             ar_send_sems, ar_recv_sems):
        my = lax.axis_index("i")
        right = (my + 1) % N_DEV

        bsem = pltpu.get_barrier_semaphore()
        for d in (1, 2, 3):
            pl.semaphore_signal(
                bsem, inc=1,
                device_id=((my + d) % N_DEV,),
                device_id_type=pl.DeviceIdType.MESH,
            )
        pl.semaphore_wait(bsem, N_DEV - 1)

        def kv_rdma(j, t, src):
            return pltpu.make_async_remote_copy(
                src_ref=src.at[0, :, j * HQ_LOCAL:(j + 1) * HQ_LOCAL, :],
                dst_ref=kv_buf.at[t],
                send_sem=kv_send_sems.at[j - 1, t],
                recv_sem=kv_recv_sems.at[t],
                device_id=(j,),
                device_id_type=pl.DeviceIdType.MESH,
            )

        @pl.when(my == 0)
        def _():
            for j in (1, 2, 3):
                for t, src in ((0, k_ref), (1, v_ref)):
                    kv_rdma(j, t, src).start()
            for t, src in ((0, k_ref), (1, v_ref)):
                pltpu.make_async_copy(
                    src.at[0, :, 0:HQ_LOCAL, :], kv_buf.at[t],
                    kv_local_sems.at[t],
                ).start()

        q = jnp.dot(x_ref[0], wq_ref[...],
                    preferred_element_type=jnp.float32)
        qb = lax.broadcasted_iota(jnp.int32, (SQ, SKV), 0) // BLK
        kb = lax.broadcasted_iota(jnp.int32, (SQ, SKV), 1) // BLK
        mask = kb <= qb

        @pl.when(my == 0)
        def _():
            for t, src in ((0, k_ref), (1, v_ref)):
                pltpu.make_async_copy(
                    src.at[0, :, 0:HQ_LOCAL, :], kv_buf.at[t],
                    kv_local_sems.at[t],
                ).wait()

        @pl.when(my != 0)
        def _():
            for t, src in ((0, k_ref), (1, v_ref)):
                kv_rdma(1, t, src).wait_recv()

        partial = jnp.zeros((SQ, D_MODEL), jnp.float32)
        for h in range(HQ_LOCAL):
            qh = q[:, h * DH:(h + 1) * DH]
            kh = kv_buf[0, :, h, :]
            vh = kv_buf[1, :, h, :]
            s = lax.dot_general(
                qh, kh, (((1,), (1,)), ((), ())),
                preferred_element_type=jnp.float32,
            ) * SCALE
            s = jnp.where(mask, s, NEG)
            m = jnp.max(s, axis=1, keepdims=True)
            e = jnp.exp(s - m)
            w = e / jnp.sum(e, axis=1, keepdims=True)
            ctx_h = jnp.dot(w, vh, preferred_element_type=jnp.float32)
            partial = partial + jnp.dot(
                ctx_h, wo_ref[h * DH:(h + 1) * DH, :],
                preferred_element_type=jnp.float32,
            )

        @pl.when(my == 0)
        def _():
            for j in (1, 2, 3):
                for t, src in ((0, k_ref), (1, v_ref)):
                    kv_rdma(j, t, src).wait_send()

        out_ref[0] = partial
        acc = partial
        for hop in range(N_DEV - 1):
            src = out_ref.at[0] if hop == 0 else comm.at[hop - 1]
            rdma = pltpu.make_async_remote_copy(
                src_ref=src,
                dst_ref=comm.at[hop],
                send_sem=ar_send_sems.at[hop],
                recv_sem=ar_recv_sems.at[hop],
                device_id=(right,),
                device_id_type=pl.DeviceIdType.MESH,
            )
            rdma.start()
            rdma.wait()
            acc = acc + comm[hop]
        out_ref[0] = acc

    return pl.pallas_call(
        body,
        out_shape=jax.ShapeDtypeStruct((1, SQ, D_MODEL), jnp.float32),
        in_specs=[
            pl.BlockSpec(memory_space=pltpu.VMEM),
            pl.BlockSpec(memory_space=pltpu.VMEM),
            pl.BlockSpec(memory_space=pl.ANY),
            pl.BlockSpec(memory_space=pl.ANY),
            pl.BlockSpec(memory_space=pltpu.VMEM),
        ],
        out_specs=pl.BlockSpec(memory_space=pltpu.VMEM),
        scratch_shapes=[
            pltpu.VMEM((2, SKV, HQ_LOCAL, DH), jnp.float32),
            pltpu.VMEM((N_DEV - 1, SQ, D_MODEL), jnp.float32),
            pltpu.SemaphoreType.DMA((N_DEV - 1, 2)),
            pltpu.SemaphoreType.DMA((2,)),
            pltpu.SemaphoreType.DMA((2,)),
            pltpu.SemaphoreType.DMA((N_DEV - 1,)),
            pltpu.SemaphoreType.DMA((N_DEV - 1,)),
        ],
        compiler_params=pltpu.CompilerParams(collective_id=0),
    )(x, Wq, K_ext, V_ext, Wo)


# device time: 295328 ns/iter; 1.2104x vs baseline; 1.2104x over previous
import jax
import jax.numpy as jnp
from jax import lax
from jax.experimental import pallas as pl
from jax.experimental.pallas import tpu as pltpu

N_DEV = 4
SQ = 1024
SKV = 1024
HQ_LOCAL = 8
DH = 128
D_MODEL = 1024
BLK = 64
SCALE = 0.08838834764831843
NEG = -1e9


def kernel(x, Wq, K_ext, V_ext, Wo):
    def body(x_ref, wq_ref, k_ref, v_ref, wo_ref, out_ref,
             kv_buf, comm,
             kv_send_sems, kv_recv_sems, kv_local_sems,
             ar_send_sems, ar_recv_sems, ag_send_sems, ag_recv_sems):
        my = lax.axis_index("i")
        right = (my + 1) % N_DEV

        bsem = pltpu.get_barrier_semaphore()
        for d in (1, 2, 3):
            pl.semaphore_signal(
                bsem, inc=1,
                device_id=((my + d) % N_DEV,),
                device_id_type=pl.DeviceIdType.MESH,
            )
        pl.semaphore_wait(bsem, N_DEV - 1)

        def kv_rdma(j, t, src):
            return pltpu.make_async_remote_copy(
                src_ref=src.at[0, :, j * HQ_LOCAL:(j + 1) * HQ_LOCAL, :],
                dst_ref=kv_buf.at[t],
                send_sem=kv_send_sems.at[j - 1, t],
                recv_sem=kv_recv_sems.at[t],
                device_id=(j,),
                device_id_type=pl.DeviceIdType.MESH,
            )

        @pl.when(my == 0)
        def _():
            for j in (1, 2, 3):
                for t, src in ((0, k_ref), (1, v_ref)):
                    kv_rdma(j, t, src).start()
            for t, src in ((0, k_ref), (1, v_ref)):
                pltpu.make_async_copy(
                    src.at[0, :, 0:HQ_LOCAL, :], kv_buf.at[t],
                    kv_local_sems.at[t],
                ).start()

        q = jnp.dot(x_ref[0], wq_ref[...],
                    preferred_element_type=jnp.float32)
        qb = lax.broadcasted_iota(jnp.int32, (SQ, SKV), 0) // BLK
        kb = lax.broadcasted_iota(jnp.int32, (SQ, SKV), 1) // BLK
        mask = kb <= qb

        @pl.when(my == 0)
        def _():
            for t, src in ((0, k_ref), (1, v_ref)):
                pltpu.make_async_copy(
                    src.at[0, :, 0:HQ_LOCAL, :], kv_buf.at[t],
                    kv_local_sems.at[t],
                ).wait()

        @pl.when(my != 0)
        def _():
            for t, src in ((0, k_ref), (1, v_ref)):
                kv_rdma(1, t, src).wait_recv()

        partial = jnp.zeros((SQ, D_MODEL), jnp.float32)
        for h in range(HQ_LOCAL):
            qh = q[:, h * DH:(h + 1) * DH]
            kh = kv_buf[0, :, h, :]
            vh = kv_buf[1, :, h, :]
            s = lax.dot_general(
                qh, kh, (((1,), (1,)), ((), ())),
                preferred_element_type=jnp.float32,
            ) * SCALE
            s = jnp.where(mask, s, NEG)
            m = jnp.max(s, axis=1, keepdims=True)
            e = jnp.exp(s - m)
            w = e / jnp.sum(e, axis=1, keepdims=True)
            ctx_h = jnp.dot(w, vh, preferred_element_type=jnp.float32)
            partial = partial + jnp.dot(
                ctx_h, wo_ref[h * DH:(h + 1) * DH, :],
                preferred_element_type=jnp.float32,
            )

        @pl.when(my == 0)
        def _():
            for j in (1, 2, 3):
                for t, src in ((0, k_ref), (1, v_ref)):
                    kv_rdma(j, t, src).wait_send()

        rows = SQ // N_DEV
        out_ref[0] = partial

        def out_chunk(c):
            return out_ref.at[0, pl.ds(c * rows, rows), :]

        for hop in range(N_DEV - 1):
            s = (my - hop) % N_DEV
            r = (my - hop - 1) % N_DEV
            rdma = pltpu.make_async_remote_copy(
                src_ref=out_chunk(s),
                dst_ref=comm.at[hop],
                send_sem=ar_send_sems.at[hop],
                recv_sem=ar_recv_sems.at[hop],
                device_id=(right,),
                device_id_type=pl.DeviceIdType.MESH,
            )
            rdma.start()
            rdma.wait_recv()
            ridx = r * rows
            out_ref[0, pl.ds(ridx, rows), :] = (
                out_ref[0, pl.ds(ridx, rows), :] + comm[hop]
            )
        for hop in range(N_DEV - 1):
            g = (my + 1 - hop) % N_DEV
            rdma = pltpu.make_async_remote_copy(
                src_ref=out_chunk(g),
                dst_ref=out_chunk(g),
                send_sem=ag_send_sems.at[hop],
                recv_sem=ag_recv_sems.at[hop],
                device_id=(right,),
                device_id_type=pl.DeviceIdType.MESH,
            )
            rdma.start()
            rdma.wait_recv()
        for hop in range(N_DEV - 1):
            pltpu.make_async_remote_copy(
                src_ref=out_chunk((my - hop) % N_DEV),
                dst_ref=comm.at[hop],
                send_sem=ar_send_sems.at[hop],
                recv_sem=ar_recv_sems.at[hop],
                device_id=(right,),
                device_id_type=pl.DeviceIdType.MESH,
            ).wait_send()
            pltpu.make_async_remote_copy(
                src_ref=out_chunk((my + 1 - hop) % N_DEV),
                dst_ref=out_chunk((my + 1 - hop) % N_DEV),
                send_sem=ag_send_sems.at[hop],
                recv_sem=ag_recv_sems.at[hop],
                device_id=(right,),
                device_id_type=pl.DeviceIdType.MESH,
            ).wait_send()

    return pl.pallas_call(
        body,
        out_shape=jax.ShapeDtypeStruct((1, SQ, D_MODEL), jnp.float32),
        in_specs=[
            pl.BlockSpec(memory_space=pltpu.VMEM),
            pl.BlockSpec(memory_space=pltpu.VMEM),
            pl.BlockSpec(memory_space=pl.ANY),
            pl.BlockSpec(memory_space=pl.ANY),
            pl.BlockSpec(memory_space=pltpu.VMEM),
        ],
        out_specs=pl.BlockSpec(memory_space=pltpu.VMEM),
        scratch_shapes=[
            pltpu.VMEM((2, SKV, HQ_LOCAL, DH), jnp.float32),
            pltpu.VMEM((N_DEV - 1, SQ // N_DEV, D_MODEL), jnp.float32),
            pltpu.SemaphoreType.DMA((N_DEV - 1, 2)),
            pltpu.SemaphoreType.DMA((2,)),
            pltpu.SemaphoreType.DMA((2,)),
            pltpu.SemaphoreType.DMA((N_DEV - 1,)),
            pltpu.SemaphoreType.DMA((N_DEV - 1,)),
            pltpu.SemaphoreType.DMA((N_DEV - 1,)),
            pltpu.SemaphoreType.DMA((N_DEV - 1,)),
        ],
        compiler_params=pltpu.CompilerParams(collective_id=0),
    )(x, Wq, K_ext, V_ext, Wo)


# device time: 211538 ns/iter; 1.6899x vs baseline; 1.3961x over previous
import jax
import jax.numpy as jnp
from jax import lax
from jax.experimental import pallas as pl
from jax.experimental.pallas import tpu as pltpu

N_DEV = 4
SQ = 1024
SKV = 1024
HQ_LOCAL = 8
DH = 128
D_MODEL = 1024
BLK = 64
SCALE = 0.08838834764831843
NEG = -1e9


def kernel(x, Wq, K_ext, V_ext, Wo):
    def body(x_ref, wq_ref, k_ref, v_ref, wo_ref, out_ref,
             kv_buf, kv_send_buf, stage, comm,
             kv_send_sems, kv_recv_sems, stage_sems,
             ar_send_sems, ar_recv_sems, ag_send_sems, ag_recv_sems):
        my = lax.axis_index("i")
        right = (my + 1) % N_DEV

        bsem = pltpu.get_barrier_semaphore()
        for d in (1, 2, 3):
            pl.semaphore_signal(
                bsem, inc=1,
                device_id=((my + d) % N_DEV,),
                device_id_type=pl.DeviceIdType.MESH,
            )
        pl.semaphore_wait(bsem, N_DEV - 1)

        def kv_rdma(j, t):
            return pltpu.make_async_remote_copy(
                src_ref=kv_send_buf.at[t, :, (j - 1) * HQ_LOCAL:j * HQ_LOCAL, :],
                dst_ref=kv_buf.at[t],
                send_sem=kv_send_sems.at[j - 1, t],
                recv_sem=kv_recv_sems.at[t],
                device_id=(j,),
                device_id_type=pl.DeviceIdType.MESH,
            )

        @pl.when(my == 0)
        def _():
            jobs = [(j, t) for j in (2, 1, 3, 0) for t in (0, 1)]

            def stage_dma(idx, slot):
                j, t = jobs[idx]
                src = k_ref if t == 0 else v_ref
                return pltpu.make_async_copy(
                    src.at[0, :, j * HQ_LOCAL:(j + 1) * HQ_LOCAL, :],
                    stage.at[slot],
                    stage_sems.at[slot],
                )

            stage_dma(0, 0).start()
            stage_dma(1, 1).start()
            for idx, (j, t) in enumerate(jobs):
                slot = idx % 2
                stage_dma(idx, slot).wait()
                bf = stage[slot].astype(jnp.bfloat16)
                if j == 0:
                    kv_buf[t] = bf
                else:
                    kv_send_buf[t, :, (j - 1) * HQ_LOCAL:j * HQ_LOCAL, :] = bf
                    kv_rdma(j, t).start()
                if idx + 2 < len(jobs):
                    stage_dma(idx + 2, slot).start()

        q = jnp.dot(x_ref[0], wq_ref[...],
                    preferred_element_type=jnp.float32)
        qb = lax.broadcasted_iota(jnp.int32, (SQ, SKV), 0) // BLK
        kb = lax.broadcasted_iota(jnp.int32, (SQ, SKV), 1) // BLK
        mask = kb <= qb

        @pl.when(my != 0)
        def _():
            for t in (0, 1):
                kv_rdma(1, t).wait_recv()

        partial = jnp.zeros((SQ, D_MODEL), jnp.float32)
        for h in range(HQ_LOCAL):
            qh = q[:, h * DH:(h + 1) * DH]
            kh = kv_buf[0, :, h, :].astype(jnp.float32)
            vh = kv_buf[1, :, h, :].astype(jnp.float32)
            s = lax.dot_general(
                qh, kh, (((1,), (1,)), ((), ())),
                preferred_element_type=jnp.float32,
            ) * SCALE
            s = jnp.where(mask, s, NEG)
            m = jnp.max(s, axis=1, keepdims=True)
            e = jnp.exp(s - m)
            w = e / jnp.sum(e, axis=1, keepdims=True)
            ctx_h = jnp.dot(w, vh, preferred_element_type=jnp.float32)
            partial = partial + jnp.dot(
                ctx_h, wo_ref[h * DH:(h + 1) * DH, :],
                preferred_element_type=jnp.float32,
            )

        @pl.when(my == 0)
        def _():
            for j in (1, 2, 3):
                for t in (0, 1):
                    kv_rdma(j, t).wait_send()

        rows = SQ // N_DEV
        out_ref[0] = partial

        def out_chunk(c):
            return out_ref.at[0, pl.ds(c * rows, rows), :]

        for hop in range(N_DEV - 1):
            s = (my - hop) % N_DEV
            r = (my - hop - 1) % N_DEV
            rdma = pltpu.make_async_remote_copy(
                src_ref=out_chunk(s),
                dst_ref=comm.at[hop],
                send_sem=ar_send_sems.at[hop],
                recv_sem=ar_recv_sems.at[hop],
                device_id=(right,),
                device_id_type=pl.DeviceIdType.MESH,
            )
            rdma.start()
            rdma.wait_recv()
            ridx = r * rows
            out_ref[0, pl.ds(ridx, rows), :] = (
                out_ref[0, pl.ds(ridx, rows), :] + comm[hop]
            )
        for hop in range(N_DEV - 1):
            g = (my + 1 - hop) % N_DEV
            rdma = pltpu.make_async_remote_copy(
                src_ref=out_chunk(g),
                dst_ref=out_chunk(g),
                send_sem=ag_send_sems.at[hop],
                recv_sem=ag_recv_sems.at[hop],
                device_id=(right,),
                device_id_type=pl.DeviceIdType.MESH,
            )
            rdma.start()
            rdma.wait_recv()
        for hop in range(N_DEV - 1):
            pltpu.make_async_remote_copy(
                src_ref=out_chunk((my - hop) % N_DEV),
                dst_ref=comm.at[hop],
                send_sem=ar_send_sems.at[hop],
                recv_sem=ar_recv_sems.at[hop],
                device_id=(right,),
                device_id_type=pl.DeviceIdType.MESH,
            ).wait_send()
            pltpu.make_async_remote_copy(
                src_ref=out_chunk((my + 1 - hop) % N_DEV),
                dst_ref=out_chunk((my + 1 - hop) % N_DEV),
                send_sem=ag_send_sems.at[hop],
                recv_sem=ag_recv_sems.at[hop],
                device_id=(right,),
                device_id_type=pl.DeviceIdType.MESH,
            ).wait_send()

    return pl.pallas_call(
        body,
        out_shape=jax.ShapeDtypeStruct((1, SQ, D_MODEL), jnp.float32),
        in_specs=[
            pl.BlockSpec(memory_space=pltpu.VMEM),
            pl.BlockSpec(memory_space=pltpu.VMEM),
            pl.BlockSpec(memory_space=pl.ANY),
            pl.BlockSpec(memory_space=pl.ANY),
            pl.BlockSpec(memory_space=pltpu.VMEM),
        ],
        out_specs=pl.BlockSpec(memory_space=pltpu.VMEM),
        scratch_shapes=[
            pltpu.VMEM((2, SKV, HQ_LOCAL, DH), jnp.bfloat16),
            pltpu.VMEM((2, SKV, 3 * HQ_LOCAL, DH), jnp.bfloat16),
            pltpu.VMEM((2, SKV, HQ_LOCAL, DH), jnp.float32),
            pltpu.VMEM((N_DEV - 1, SQ // N_DEV, D_MODEL), jnp.float32),
            pltpu.SemaphoreType.DMA((N_DEV - 1, 2)),
            pltpu.SemaphoreType.DMA((2,)),
            pltpu.SemaphoreType.DMA((2,)),
            pltpu.SemaphoreType.DMA((N_DEV - 1,)),
            pltpu.SemaphoreType.DMA((N_DEV - 1,)),
            pltpu.SemaphoreType.DMA((N_DEV - 1,)),
            pltpu.SemaphoreType.DMA((N_DEV - 1,)),
        ],
        compiler_params=pltpu.CompilerParams(
            collective_id=0,
            vmem_limit_bytes=60 * 1024 * 1024,
        ),
    )(x, Wq, K_ext, V_ext, Wo)


# device time: 178242 ns/iter; 2.0055x vs baseline; 1.1868x over previous
import jax
import jax.numpy as jnp
from jax import lax
from jax.experimental import pallas as pl
from jax.experimental.pallas import tpu as pltpu

N_DEV = 4
SQ = 1024
SKV = 1024
HQ_LOCAL = 8
DH = 128
D_MODEL = 1024
BLK = 64
SCALE = 0.08838834764831843
NEG = -1e9


def kernel(x, Wq, K_ext, V_ext, Wo):
    def body(x_ref, wq_ref, k_ref, v_ref, wo_ref, out_ref,
             kv_buf, kv_send_buf, stage, comm, ar_bf, ag_comm, ag0_bf,
             kv_send_sems, kv_recv_sems, stage_sems,
             ar_send_sems, ar_recv_sems, ag_send_sems, ag_recv_sems):
        my = lax.axis_index("i")
        right = (my + 1) % N_DEV

        bsem = pltpu.get_barrier_semaphore()
        for d in (1, 2, 3):
            pl.semaphore_signal(
                bsem, inc=1,
                device_id=((my + d) % N_DEV,),
                device_id_type=pl.DeviceIdType.MESH,
            )
        pl.semaphore_wait(bsem, N_DEV - 1)

        def kv_rdma(j, t):
            return pltpu.make_async_remote_copy(
                src_ref=kv_send_buf.at[t, :, (j - 1) * HQ_LOCAL:j * HQ_LOCAL, :],
                dst_ref=kv_buf.at[t],
                send_sem=kv_send_sems.at[j - 1, t],
                recv_sem=kv_recv_sems.at[t],
                device_id=(j,),
                device_id_type=pl.DeviceIdType.MESH,
            )

        @pl.when(my == 0)
        def _():
            jobs = [(j, t) for j in (2, 1, 3, 0) for t in (0, 1)]

            def stage_dma(idx, slot):
                j, t = jobs[idx]
                src = k_ref if t == 0 else v_ref
                return pltpu.make_async_copy(
                    src.at[0, :, j * HQ_LOCAL:(j + 1) * HQ_LOCAL, :],
                    stage.at[slot],
                    stage_sems.at[slot],
                )

            stage_dma(0, 0).start()
            stage_dma(1, 1).start()
            for idx, (j, t) in enumerate(jobs):
                slot = idx % 2
                stage_dma(idx, slot).wait()
                bf = stage[slot].astype(jnp.bfloat16)
                if j == 0:
                    kv_buf[t] = bf
                else:
                    kv_send_buf[t, :, (j - 1) * HQ_LOCAL:j * HQ_LOCAL, :] = bf
                    kv_rdma(j, t).start()
                if idx + 2 < len(jobs):
                    stage_dma(idx + 2, slot).start()

        q = jnp.dot(x_ref[0], wq_ref[...],
                    preferred_element_type=jnp.float32)
        qb = lax.broadcasted_iota(jnp.int32, (SQ, SKV), 0) // BLK
        kb = lax.broadcasted_iota(jnp.int32, (SQ, SKV), 1) // BLK
        mask = kb <= qb

        @pl.when(my != 0)
        def _():
            for t in (0, 1):
                kv_rdma(1, t).wait_recv()

        partial = jnp.zeros((SQ, D_MODEL), jnp.float32)
        for h in range(HQ_LOCAL):
            qh = q[:, h * DH:(h + 1) * DH]
            kh = kv_buf[0, :, h, :].astype(jnp.float32)
            vh = kv_buf[1, :, h, :].astype(jnp.float32)
            s = lax.dot_general(
                qh, kh, (((1,), (1,)), ((), ())),
                preferred_element_type=jnp.float32,
            ) * SCALE
            s = jnp.where(mask, s, NEG)
            m = jnp.max(s, axis=1, keepdims=True)
            e = jnp.exp(s - m)
            w = e / jnp.sum(e, axis=1, keepdims=True)
            ctx_h = jnp.dot(w, vh, preferred_element_type=jnp.float32)
            partial = partial + jnp.dot(
                ctx_h, wo_ref[h * DH:(h + 1) * DH, :],
                preferred_element_type=jnp.float32,
            )

        @pl.when(my == 0)
        def _():
            for j in (1, 2, 3):
                for t in (0, 1):
                    kv_rdma(j, t).wait_send()

        rows = SQ // N_DEV
        out_ref[0] = partial

        def out_chunk(c):
            return out_ref.at[0, pl.ds(c * rows, rows), :]

        def rs_rdma(hop):
            return pltpu.make_async_remote_copy(
                src_ref=ar_bf.at[hop],
                dst_ref=comm.at[hop],
                send_sem=ar_send_sems.at[hop],
                recv_sem=ar_recv_sems.at[hop],
                device_id=(right,),
                device_id_type=pl.DeviceIdType.MESH,
            )

        def ag_rdma(hop):
            return pltpu.make_async_remote_copy(
                src_ref=ag0_bf.at[0] if hop == 0 else ag_comm.at[hop - 1],
                dst_ref=ag_comm.at[hop],
                send_sem=ag_send_sems.at[hop],
                recv_sem=ag_recv_sems.at[hop],
                device_id=(right,),
                device_id_type=pl.DeviceIdType.MESH,
            )

        for hop in range(N_DEV - 1):
            s = (my - hop) % N_DEV
            r = (my - hop - 1) % N_DEV
            ar_bf[hop] = out_ref[0, pl.ds(s * rows, rows), :].astype(jnp.bfloat16)
            rdma = rs_rdma(hop)
            rdma.start()
            rdma.wait_recv()
            ridx = r * rows
            out_ref[0, pl.ds(ridx, rows), :] = (
                out_ref[0, pl.ds(ridx, rows), :]
                + comm[hop].astype(jnp.float32)
            )
        ag0_bf[0] = out_ref[
            0, pl.ds(((my + 1) % N_DEV) * rows, rows), :
        ].astype(jnp.bfloat16)
        for hop in range(N_DEV - 1):
            g = (my - hop) % N_DEV
            rdma = ag_rdma(hop)
            rdma.start()
            rdma.wait_recv()
            out_ref[0, pl.ds(g * rows, rows), :] = (
                ag_comm[hop].astype(jnp.float32)
            )
        for hop in range(N_DEV - 1):
            rs_rdma(hop).wait_send()
            ag_rdma(hop).wait_send()

    return pl.pallas_call(
        body,
        out_shape=jax.ShapeDtypeStruct((1, SQ, D_MODEL), jnp.float32),
        in_specs=[
            pl.BlockSpec(memory_space=pltpu.VMEM),
            pl.BlockSpec(memory_space=pltpu.VMEM),
            pl.BlockSpec(memory_space=pl.ANY),
            pl.BlockSpec(memory_space=pl.ANY),
            pl.BlockSpec(memory_space=pltpu.VMEM),
        ],
        out_specs=pl.BlockSpec(memory_space=pltpu.VMEM),
        scratch_shapes=[
            pltpu.VMEM((2, SKV, HQ_LOCAL, DH), jnp.bfloat16),
            pltpu.VMEM((2, SKV, 3 * HQ_LOCAL, DH), jnp.bfloat16),
            pltpu.VMEM((2, SKV, HQ_LOCAL, DH), jnp.float32),
            pltpu.VMEM((N_DEV - 1, SQ // N_DEV, D_MODEL), jnp.bfloat16),
            pltpu.VMEM((N_DEV - 1, SQ // N_DEV, D_MODEL), jnp.bfloat16),
            pltpu.VMEM((N_DEV - 1, SQ // N_DEV, D_MODEL), jnp.bfloat16),
            pltpu.VMEM((1, SQ // N_DEV, D_MODEL), jnp.bfloat16),
            pltpu.SemaphoreType.DMA((N_DEV - 1, 2)),
            pltpu.SemaphoreType.DMA((2,)),
            pltpu.SemaphoreType.DMA((2,)),
            pltpu.SemaphoreType.DMA((N_DEV - 1,)),
            pltpu.SemaphoreType.DMA((N_DEV - 1,)),
            pltpu.SemaphoreType.DMA((N_DEV - 1,)),
            pltpu.SemaphoreType.DMA((N_DEV - 1,)),
        ],
        compiler_params=pltpu.CompilerParams(
            collective_id=0,
            vmem_limit_bytes=63 * 1024 * 1024,
        ),
    )(x, Wq, K_ext, V_ext, Wo)
